# baseline (speedup 1.0000x reference)
import math

import jax
import jax.numpy as jnp
from jax.experimental import pallas as pl
from jax.experimental.pallas import tpu as pltpu

RES = 0.1
B = 1024
U = 8


def _levels(grids):
    return [(g.shape[0], g.shape[1]) for g in grids]


def _body(a_ref, sv_ref, t_ref, o_ref, s0_ref, s1_ref, *, hw, nlev):
    for l in range(nlev):
        W = hw[l][1]

        def gather(i, _):
            for u in range(U):
                k = i * U + u
                a = a_ref[0, 0, l * B + k]
                r0 = a >> 7
                s0_ref[k] = t_ref[pl.ds(r0, 2), 0, :]
                r1 = (a + W) >> 7
                s1_ref[k] = t_ref[pl.ds(r1, 2), 0, :]
            return 0

        jax.lax.fori_loop(0, B // U, gather, 0)

        c0a = s0_ref[:, 0, :]
        c0b = s0_ref[:, 1, :]
        c1a = s1_ref[:, 0, :]
        c1b = s1_ref[:, 1, :]

        iota = jax.lax.broadcasted_iota(jnp.int32, (B, 128), 1).astype(jnp.float32)
        s0 = sv_ref[0, :, l : l + 1]
        s1 = sv_ref[0, :, 8 + l : 9 + l]
        wy = sv_ref[0, :, 16 + l : 17 + l]

        def tent(s):
            return jnp.maximum(1.0 - jnp.abs(iota - s), 0.0)

        row0 = c0a * tent(s0) + c0b * tent(s0 - 128.0)
        row1 = c1a * tent(s1) + c1b * tent(s1 - 128.0)
        val = jnp.sum(row0 * (1.0 - wy) + row1 * wy, axis=1)
        o_ref[:, l : l + 1] = val[:, None]


def kernel(x, g0, g1, g2, g3, g4, g5, g6, g7):
    grids = [g0, g1, g2, g3, g4, g5, g6, g7]
    hw = _levels(grids)
    nlev = len(grids)
    n = x.shape[0]
    nblk = (n + B - 1) // B
    npad = nblk * B

    sizes = [h * w for h, w in hw]
    offs = [0]
    for s in sizes:
        offs.append(offs[-1] + s)
    total = offs[-1]
    rows = (total + 512 + 127) // 128 + 1
    flat = jnp.concatenate(
        [g.reshape(-1) for g in grids]
        + [jnp.zeros((rows * 128 - total,), jnp.float32)]
    )
    table = flat.reshape(rows, 1, 128)

    xc = jax.lax.stop_gradient(x)
    lat_idx = jnp.round((90.0 - xc[:, 0]) / RES).astype(jnp.int32)
    lon_idx = jnp.round(xc[:, 1] / RES).astype(jnp.int32)
    H_out, W_out = 1801, 3600

    a0s, s0s, s1s, wys = [], [], [], []
    for l in range(nlev):
        Hin, Win = hw[l]
        sy = jnp.maximum((lat_idx.astype(jnp.float32) + 0.5) * (Hin / H_out) - 0.5, 0.0)
        sx = jnp.maximum((lon_idx.astype(jnp.float32) + 0.5) * (Win / W_out) - 0.5, 0.0)
        y0 = jnp.floor(sy).astype(jnp.int32)
        x0 = jnp.floor(sx).astype(jnp.int32)
        y0 = jnp.minimum(y0, Hin - 2)
        x0 = jnp.minimum(x0, Win - 2)
        wy = jnp.clip(sy - y0.astype(jnp.float32), 0.0, 1.0)
        wx = jnp.clip(sx - x0.astype(jnp.float32), 0.0, 1.0)
        a0 = offs[l] + y0 * Win + x0
        a1 = a0 + Win
        a0s.append(a0)
        s0s.append((a0 & 127).astype(jnp.float32) + wx)
        s1s.append((a1 & 127).astype(jnp.float32) + wx)
        wys.append(wy)

    def blockify(cols):
        arr = jnp.stack(cols, axis=0)
        arr = jnp.pad(arr, ((0, 0), (0, npad - n)))
        return arr.reshape(arr.shape[0], nblk, B).transpose(1, 2, 0)

    A = jnp.stack(a0s, axis=0)
    A = jnp.pad(A, ((0, 0), (0, npad - n)))
    A = A.reshape(nlev, nblk, B).transpose(1, 0, 2).reshape(nblk, 1, nlev * B)

    SV = jnp.concatenate(
        [blockify(s0s), blockify(s1s), blockify(wys)], axis=2
    )

    body = lambda a_ref, sv_ref, t_ref, o_ref, s0_ref, s1_ref: _body(
        a_ref, sv_ref, t_ref, o_ref, s0_ref, s1_ref, hw=hw, nlev=nlev
    )

    out = pl.pallas_call(
        body,
        grid=(nblk,),
        in_specs=[
            pl.BlockSpec((1, 1, nlev * B), lambda b: (b, 0, 0), memory_space=pltpu.SMEM),
            pl.BlockSpec((1, B, 3 * nlev), lambda b: (b, 0, 0)),
            pl.BlockSpec((rows, 1, 128), lambda b: (0, 0, 0)),
        ],
        out_specs=pl.BlockSpec((B, nlev), lambda b: (b, 0)),
        out_shape=jax.ShapeDtypeStruct((npad, nlev), jnp.float32),
        scratch_shapes=[
            pltpu.VMEM((B, 2, 128), jnp.float32),
            pltpu.VMEM((B, 2, 128), jnp.float32),
        ],
        compiler_params=pltpu.CompilerParams(
            dimension_semantics=("parallel",),
        ),
    )(A, SV, table)
    return out[:n]


# speedup vs baseline: 1.0424x; 1.0424x over previous
import math

import jax
import jax.numpy as jnp
from jax.experimental import pallas as pl
from jax.experimental.pallas import tpu as pltpu

RES = 0.1
B = 1024
U = 8


def _levels(grids):
    return [(g.shape[0], g.shape[1]) for g in grids]


def _body(a_ref, sv_ref, t_ref, o_ref, s0a_ref, s0b_ref, s1a_ref, s1b_ref, *, hw, nlev):
    for l in range(nlev):
        W = hw[l][1]
        small = W <= 127

        def gather(i, _):
            for u in range(U):
                k = i * U + u
                a = a_ref[0, 0, l * B + k]
                r0 = a >> 7
                s0a_ref[k] = t_ref[pl.ds(r0, 1), 0, :]
                s0b_ref[k] = t_ref[pl.ds(r0 + 1, 1), 0, :]
                if not small:
                    r1 = (a + W) >> 7
                    s1a_ref[k] = t_ref[pl.ds(r1, 1), 0, :]
                    s1b_ref[k] = t_ref[pl.ds(r1 + 1, 1), 0, :]
            return 0

        jax.lax.fori_loop(0, B // U, gather, 0)

        c0a = s0a_ref[:, 0, :]
        c0b = s0b_ref[:, 0, :]
        c1a = c0a if small else s1a_ref[:, 0, :]
        c1b = c0b if small else s1b_ref[:, 0, :]

        iota = jax.lax.broadcasted_iota(jnp.int32, (B, 128), 1).astype(jnp.float32)
        s0 = sv_ref[0, :, l : l + 1]
        s1 = sv_ref[0, :, 8 + l : 9 + l]
        wy = sv_ref[0, :, 16 + l : 17 + l]

        def tent(s):
            return jnp.maximum(1.0 - jnp.abs(iota - s), 0.0)

        row0 = c0a * tent(s0) + c0b * tent(s0 - 128.0)
        row1 = c1a * tent(s1) + c1b * tent(s1 - 128.0)
        val = jnp.sum(row0 * (1.0 - wy) + row1 * wy, axis=1)
        o_ref[:, l : l + 1] = val[:, None]


def kernel(x, g0, g1, g2, g3, g4, g5, g6, g7):
    grids = [g0, g1, g2, g3, g4, g5, g6, g7]
    hw = _levels(grids)
    nlev = len(grids)
    n = x.shape[0]
    nblk = (n + B - 1) // B
    npad = nblk * B

    sizes = [h * w for h, w in hw]
    offs = [0]
    for s in sizes:
        offs.append(offs[-1] + s)
    total = offs[-1]
    rows = (total + 512 + 127) // 128 + 1
    flat = jnp.concatenate(
        [g.reshape(-1) for g in grids]
        + [jnp.zeros((rows * 128 - total,), jnp.float32)]
    )
    table = flat.reshape(rows, 1, 128)

    xc = jax.lax.stop_gradient(x)
    lat_idx = jnp.round((90.0 - xc[:, 0]) / RES).astype(jnp.int32)
    lon_idx = jnp.round(xc[:, 1] / RES).astype(jnp.int32)
    H_out, W_out = 1801, 3600

    a0s, s0s, s1s, wys = [], [], [], []
    for l in range(nlev):
        Hin, Win = hw[l]
        sy = jnp.maximum((lat_idx.astype(jnp.float32) + 0.5) * (Hin / H_out) - 0.5, 0.0)
        sx = jnp.maximum((lon_idx.astype(jnp.float32) + 0.5) * (Win / W_out) - 0.5, 0.0)
        y0 = jnp.floor(sy).astype(jnp.int32)
        x0 = jnp.floor(sx).astype(jnp.int32)
        y0 = jnp.minimum(y0, Hin - 2)
        x0 = jnp.minimum(x0, Win - 2)
        wy = jnp.clip(sy - y0.astype(jnp.float32), 0.0, 1.0)
        wx = jnp.clip(sx - x0.astype(jnp.float32), 0.0, 1.0)
        a0 = offs[l] + y0 * Win + x0
        a0s.append(a0)
        s0s.append((a0 & 127).astype(jnp.float32) + wx)
        if Win <= 127:
            s1s.append((a0 & 127).astype(jnp.float32) + Win + wx)
        else:
            s1s.append(((a0 + Win) & 127).astype(jnp.float32) + wx)
        wys.append(wy)

    def blockify(cols):
        arr = jnp.stack(cols, axis=0)
        arr = jnp.pad(arr, ((0, 0), (0, npad - n)))
        return arr.reshape(arr.shape[0], nblk, B).transpose(1, 2, 0)

    A = jnp.stack(a0s, axis=0)
    A = jnp.pad(A, ((0, 0), (0, npad - n)))
    A = A.reshape(nlev, nblk, B).transpose(1, 0, 2).reshape(nblk, 1, nlev * B)

    SV = jnp.concatenate(
        [blockify(s0s), blockify(s1s), blockify(wys)], axis=2
    )

    body = lambda a_ref, sv_ref, t_ref, o_ref, sa, sb, sc, sd: _body(
        a_ref, sv_ref, t_ref, o_ref, sa, sb, sc, sd, hw=hw, nlev=nlev
    )

    out = pl.pallas_call(
        body,
        grid=(nblk,),
        in_specs=[
            pl.BlockSpec((1, 1, nlev * B), lambda b: (b, 0, 0), memory_space=pltpu.SMEM),
            pl.BlockSpec((1, B, 3 * nlev), lambda b: (b, 0, 0)),
            pl.BlockSpec((rows, 1, 128), lambda b: (0, 0, 0)),
        ],
        out_specs=pl.BlockSpec((B, nlev), lambda b: (b, 0)),
        out_shape=jax.ShapeDtypeStruct((npad, nlev), jnp.float32),
        scratch_shapes=[
            pltpu.VMEM((B, 1, 128), jnp.float32),
            pltpu.VMEM((B, 1, 128), jnp.float32),
            pltpu.VMEM((B, 1, 128), jnp.float32),
            pltpu.VMEM((B, 1, 128), jnp.float32),
        ],
        compiler_params=pltpu.CompilerParams(
            dimension_semantics=("parallel",),
        ),
    )(A, SV, table)
    return out[:n]
